# baseline (speedup 1.0000x reference)
import functools

import jax
import jax.numpy as jnp
from jax import lax
from jax.experimental import pallas as pl
from jax.experimental.pallas import tpu as pltpu
from jax.experimental.pallas import tpu_sc as plsc

NC = 2
NS = 16
NW = NC * NS


@functools.lru_cache(maxsize=None)
def _build_gather(total, vocab, dim, n_chunks, n_buf):
    b_per_w = total // NW
    chunk = b_per_w // n_chunks
    mesh = plsc.VectorSubcoreMesh(core_axis_name="c", subcore_axis_name="s")

    @functools.partial(
        pl.kernel,
        mesh=mesh,
        out_type=jax.ShapeDtypeStruct((total, dim), jnp.float32),
        scratch_types=[
            pltpu.VMEM((b_per_w,), jnp.int32),
            pltpu.VMEM((n_buf, chunk, dim), jnp.float32),
            pltpu.SemaphoreType.DMA,
            pltpu.SemaphoreType.DMA,
        ],
        compiler_params=pltpu.CompilerParams(use_tc_tiling_on_sc=False),
    )
    def gather_kernel(table_hbm, idx_hbm, out_hbm, idx_v, rows_v, gsem, wsem):
        wid = lax.axis_index("s") * NC + lax.axis_index("c")
        base = wid * b_per_w

        def gather(c):
            return pltpu.async_copy(
                table_hbm.at[idx_v.at[pl.ds(c * chunk, chunk)]],
                rows_v.at[c % n_buf], gsem)

        def write(c):
            return pltpu.async_copy(
                rows_v.at[c % n_buf],
                out_hbm.at[pl.ds(base + c * chunk, chunk)], wsem)

        pltpu.sync_copy(idx_hbm.at[pl.ds(base, b_per_w)], idx_v)
        gs = {}
        ws = {}
        waited = set()
        for c in range(min(n_buf - 1, n_chunks)):
            gs[c] = gather(c)
        for c in range(n_chunks):
            gs[c].wait()
            ws[c] = write(c)
            n = c + n_buf - 1
            if n < n_chunks:
                prev = n - n_buf
                if prev >= 0:
                    ws[prev].wait()
                    waited.add(prev)
                gs[n] = gather(n)
        for c in range(n_chunks):
            if c not in waited:
                ws[c].wait()

    return gather_kernel


def kernel(indices, table):
    batch, fields = indices.shape
    vocab, dim = table.shape
    total = batch * fields
    idx_flat = indices.reshape(total).astype(jnp.int32)
    gather = _build_gather(total, vocab, dim, n_chunks=16, n_buf=4)
    out = gather(table, idx_flat)
    return out.reshape(batch, fields, dim)
